# baseline (speedup 1.0000x reference)
import functools

import jax
import jax.numpy as jnp
import numpy as np
from jax import lax
from jax.experimental import pallas as pl
from jax.experimental.pallas import tpu as pltpu
from jax.experimental.pallas import tpu_sc as plsc

_N = 10000
_NT = 10240
_E = 160000
_EP = 163840
_NCHUNK = 1280
_CPT = 80
_D = 512
_F = 128
_BR = 1000
_EPS = 1e-8

_SC_MESH = plsc.VectorSubcoreMesh(
    core_axis_name="c", subcore_axis_name="s", num_cores=2, num_subcores=16)


@functools.partial(
    pl.kernel,
    out_type=[jax.ShapeDtypeStruct((_NT, 16), jnp.float32),
              jax.ShapeDtypeStruct((_NT, 16), jnp.float32)],
    mesh=_SC_MESH,
    scratch_types=[
        pltpu.VMEM_SHARED((_NT, 16), jnp.float32),
        pltpu.VMEM((640, 16), jnp.float32),
        pltpu.VMEM((128, 16), jnp.float32),
        pltpu.VMEM((_CPT, 128), jnp.int32),
    ],
)
def _sc_degrees(sd_ref, dd_ref, dsrc_ref, ddst_ref, acc, zb, ones, idxv):
    c = lax.axis_index("c")
    s = lax.axis_index("s")

    def _zrow(i, _):
        zb[i, :] = jnp.zeros((16,), jnp.float32)
        return 0
    lax.fori_loop(0, 640, _zrow, 0)

    def _orow(i, _):
        ones[i, :] = jnp.full((16,), 1.0, jnp.float32)
        return 0
    lax.fori_loop(0, 128, _orow, 0)

    for half in range(2):
        idx_src = sd_ref if half == 0 else dd_ref
        out = dsrc_ref if half == 0 else ddst_ref

        @pl.when(c == half)
        def _():
            pltpu.sync_copy(zb, acc.at[pl.ds(s * 640, 640)])
            pltpu.sync_copy(idx_src.at[pl.ds(s * _CPT, _CPT)], idxv)

        plsc.subcore_barrier()

        @pl.when(c == half)
        def _():
            def _scat(j, _):
                pltpu.sync_copy(ones, acc.at[idxv.at[j]], add=True)
                return 0
            lax.fori_loop(0, _CPT, _scat, 0)

        plsc.subcore_barrier()

        @pl.when(c == half)
        def _():
            pltpu.sync_copy(acc.at[pl.ds(s * 640, 640)],
                            out.at[pl.ds(s * 640, 640)])


@functools.partial(
    pl.kernel,
    out_type=[jax.ShapeDtypeStruct((_NT, _F), jnp.float32)] * 4,
    mesh=_SC_MESH,
    scratch_types=[
        pltpu.VMEM_SHARED((_NT, _F), jnp.float32),
        pltpu.VMEM((_CPT * 128,), jnp.int32),
        pltpu.VMEM((4, 128), jnp.int32),
        pltpu.VMEM((2, 128, _F), jnp.float32),
        pltpu.VMEM((128,), jnp.int32),
        pltpu.SemaphoreType.DMA,
        pltpu.SemaphoreType.DMA,
        pltpu.SemaphoreType.DMA,
    ],
)
def _sc_propagate(t0, t1, t2, t3, sp_ref, dp_ref, cnt_ref,
                  o0, o1, o2, o3, acc, sidx, dring, rows, cntv,
                  gsem, dsem, ssem):
    c = lax.axis_index("c")
    s = lax.axis_index("s")

    pltpu.sync_copy(sp_ref.at[pl.ds(s * (_CPT * 128), _CPT * 128)], sidx)
    pltpu.sync_copy(cnt_ref, cntv)
    ncha = cntv[pl.ds(0, 16)][0]
    nloc = jnp.minimum(_CPT, jnp.maximum(0, (ncha - s + 15) // 16))

    tables = (t0, t1, t2, t3)
    outs = (o0, o1, o2, o3)
    for half in range(2):
        for b in range(2):
            tbl = tables[half * 2 + b]
            outb = outs[half * 2 + b]

            @pl.when(c == half)
            def _():
                def _zrow(i, _):
                    for k in range(8):
                        rows[0, i, pl.ds(k * 16, 16)] = (
                            jnp.zeros((16,), jnp.float32))
                    return 0
                lax.fori_loop(0, 128, _zrow, 0)
                for t in range(5):
                    pltpu.sync_copy(
                        rows.at[0], acc.at[pl.ds(s * 640 + t * 128, 128)])

            plsc.subcore_barrier()

            @pl.when(jnp.logical_and(c == half, nloc > 0))
            def _():
                @pl.when(nloc > 1)
                def _():
                    pltpu.async_copy(
                        dp_ref.at[pl.ds((s * _CPT + 1) * 128, 128)],
                        dring.at[1], dsem)

                pltpu.async_copy(
                    dp_ref.at[pl.ds(s * _CPT * 128, 128)], dring.at[0], dsem)
                pltpu.async_copy(
                    tbl.at[sidx.at[pl.ds(0, 128)]], rows.at[0], gsem)

                def _step(jj, _):
                    for bb in range(2):
                        cc = 2 * jj + bb

                        @pl.when(cc < nloc)
                        def _():
                            slot = lax.rem(cc, 4)

                            @pl.when(cc >= 1)
                            def _():
                                pltpu.make_async_copy(
                                    rows.at[1 - bb], acc.at[dring.at[slot]],
                                    ssem).wait()

                            @pl.when(cc + 1 < nloc)
                            def _():
                                pltpu.async_copy(
                                    tbl.at[sidx.at[pl.ds((cc + 1) * 128,
                                                         128)]],
                                    rows.at[1 - bb], gsem)

                            pltpu.make_async_copy(
                                dp_ref.at[pl.ds((s * _CPT + cc) * 128, 128)],
                                dring.at[slot], dsem).wait()
                            pltpu.make_async_copy(
                                tbl.at[sidx.at[pl.ds(cc * 128, 128)]],
                                rows.at[bb], gsem).wait()
                            pltpu.async_copy(
                                rows.at[bb], acc.at[dring.at[slot]],
                                ssem, add=True)

                            @pl.when(cc + 2 < nloc)
                            def _():
                                slot2 = lax.rem(cc + 2, 4)
                                pltpu.async_copy(
                                    dp_ref.at[pl.ds((s * _CPT + cc + 2) * 128,
                                                    128)],
                                    dring.at[slot2], dsem)
                    return 0
                lax.fori_loop(0, (nloc + 1) // 2, _step, 0)
                pltpu.make_async_copy(
                    rows.at[lax.rem(nloc - 1, 2)],
                    acc.at[dring.at[lax.rem(nloc - 1, 4)]], ssem).wait()

            plsc.subcore_barrier()

            @pl.when(c == half)
            def _():
                pltpu.sync_copy(acc.at[pl.ds(s * 640, 640)],
                                outb.at[pl.ds(s * 640, 640)])

            plsc.subcore_barrier()


def _row(i):
    return (i, 0)


def _const(i):
    return (0, 0)


def _prep_body(x_ref, m_ref, tok_ref, ds_ref, dd_ref,
               t_refs0, t_refs1, t_refs2, t_refs3, ns_ref, nd_ref, s2_ref):
    ns = lax.rsqrt(jnp.maximum(ds_ref[:, 0:1], 1.0))
    nd = lax.rsqrt(jnp.maximum(dd_ref[:, 0:1], 1.0))
    m = m_ref[...]
    xm = jnp.where(m > 0.5, tok_ref[...], x_ref[...])
    t0 = xm * ns
    for k, ref in enumerate((t_refs0, t_refs1, t_refs2, t_refs3)):
        ref[...] = t0[:, k * _F:(k + 1) * _F]
    ns_ref[...] = ns
    nd_ref[...] = nd
    s2_ref[...] = jnp.where(m > 0.5, 0.0, ns)


def _prep(x, mf, tok, dsrc, ddst):
    return pl.pallas_call(
        _prep_body,
        grid=(_N // _BR,),
        in_specs=[
            pl.BlockSpec((_BR, _D), _row),
            pl.BlockSpec((_BR, 1), _row),
            pl.BlockSpec((1, _D), _const),
            pl.BlockSpec((_BR, 16), _row),
            pl.BlockSpec((_BR, 16), _row),
        ],
        out_specs=[pl.BlockSpec((_BR, _F), _row)] * 4 + [
            pl.BlockSpec((_BR, 1), _row)] * 3,
        out_shape=[jax.ShapeDtypeStruct((_N, _F), jnp.float32)] * 4 + [
            jax.ShapeDtypeStruct((_N, 1), jnp.float32)] * 3,
    )(x, mf, tok, dsrc, ddst)


def _mm1_body(a0, a1, a2, a3, nd_ref, ns_ref, w_ref,
              o0, o1, o2, o3):
    a = jnp.concatenate([a0[...], a1[...], a2[...], a3[...]], axis=1)
    h = jnp.dot(a * nd_ref[...], w_ref[...],
                preferred_element_type=jnp.float32)
    t = jax.nn.relu(h) * ns_ref[...]
    for k, ref in enumerate((o0, o1, o2, o3)):
        ref[...] = t[:, k * _F:(k + 1) * _F]


def _mm1(aggs, nd, ns, w):
    return pl.pallas_call(
        _mm1_body,
        grid=(_N // _BR,),
        in_specs=[pl.BlockSpec((_BR, _F), _row)] * 4 + [
            pl.BlockSpec((_BR, 1), _row),
            pl.BlockSpec((_BR, 1), _row),
            pl.BlockSpec((_D, _D), _const),
        ],
        out_specs=[pl.BlockSpec((_BR, _F), _row)] * 4,
        out_shape=[jax.ShapeDtypeStruct((_N, _F), jnp.float32)] * 4,
    )(*aggs, nd, ns, w)


def _mm2_body(a0, a1, a2, a3, nd_ref, s2_ref, w1_ref, w2_ref,
              o0, o1, o2, o3):
    a = jnp.concatenate([a0[...], a1[...], a2[...], a3[...]], axis=1)
    h = jnp.dot(a * nd_ref[...], w1_ref[...],
                preferred_element_type=jnp.float32)
    h = jax.nn.relu(h)
    rep = jnp.dot(h, w2_ref[...], preferred_element_type=jnp.float32)
    t = rep * s2_ref[...]
    for k, ref in enumerate((o0, o1, o2, o3)):
        ref[...] = t[:, k * _F:(k + 1) * _F]


def _mm2(aggs, nd, s2, w1, w2):
    return pl.pallas_call(
        _mm2_body,
        grid=(_N // _BR,),
        in_specs=[pl.BlockSpec((_BR, _F), _row)] * 4 + [
            pl.BlockSpec((_BR, 1), _row),
            pl.BlockSpec((_BR, 1), _row),
            pl.BlockSpec((_D, _D), _const),
            pl.BlockSpec((_D, _D), _const),
        ],
        out_specs=[pl.BlockSpec((_BR, _F), _row)] * 4,
        out_shape=[jax.ShapeDtypeStruct((_N, _F), jnp.float32)] * 4,
    )(*aggs, nd, s2, w1, w2)


def _final_body(a0, a1, a2, a3, nd_ref, w_ref, x_ref, m_ref, o_ref, acc_ref):
    i = pl.program_id(0)

    @pl.when(i == 0)
    def _():
        acc_ref[0] = 0.0
        acc_ref[1] = 0.0

    a = jnp.concatenate([a0[...], a1[...], a2[...], a3[...]], axis=1)
    r = jnp.dot(a * nd_ref[...], w_ref[...],
                preferred_element_type=jnp.float32)
    x = x_ref[...]
    xr = jnp.sum(x * r, axis=1, keepdims=True)
    xx = jnp.sum(x * x, axis=1, keepdims=True)
    rr = jnp.sum(r * r, axis=1, keepdims=True)
    cos = xr / ((jnp.sqrt(xx) + _EPS) * (jnp.sqrt(rr) + _EPS))
    per = (1.0 - cos) ** 2
    m = m_ref[...]
    acc_ref[0] += jnp.sum(per * m)
    acc_ref[1] += jnp.sum(m)

    @pl.when(i == pl.num_programs(0) - 1)
    def _():
        o_ref[...] = jnp.reshape(acc_ref[0] / jnp.maximum(acc_ref[1], 1.0),
                                 (1, 1))


def _final(aggs, nd, w, x, mf):
    return pl.pallas_call(
        _final_body,
        grid=(_N // _BR,),
        in_specs=[pl.BlockSpec((_BR, _F), _row)] * 4 + [
            pl.BlockSpec((_BR, 1), _row),
            pl.BlockSpec((_D, _D), _const),
            pl.BlockSpec((_BR, _D), _row),
            pl.BlockSpec((_BR, 1), _row),
        ],
        out_specs=pl.BlockSpec((1, 1), _const),
        out_shape=jax.ShapeDtypeStruct((1, 1), jnp.float32),
        scratch_shapes=[pltpu.SMEM((2,), jnp.float32)],
    )(*aggs, nd, w, x, mf)


_KMAP = np.array([(d % _CPT) * 16 + d // _CPT for d in range(_NCHUNK)],
                 dtype=np.int32)


def _interleave(a):
    return a.reshape(_NCHUNK, 128)[_KMAP].reshape(-1)


_ER = _E // 128
_TRIU128 = np.triu(np.ones((128, 128), np.float32))
_TRIL_R = np.tril(np.ones((_ER, _ER), np.float32))


def _compact(keep, src, dst, fill_src, fill_dst):
    k2 = keep.astype(jnp.float32).reshape(_ER, 128)
    within = jnp.dot(k2, _TRIU128)
    rowsums = within[:, 127]
    rowincl = jnp.dot(_TRIL_R, rowsums)
    pos_incl = ((rowincl - rowsums)[:, None] + within).astype(
        jnp.int32).reshape(-1)
    cnt = rowincl[_ER - 1].astype(jnp.int32)
    idx = jnp.where(keep, pos_incl - 1, _EP)
    comb = jnp.zeros((_EP,), jnp.int32).at[idx].add(
        jnp.left_shift(src, 16) | dst, mode='drop')
    valid = jnp.arange(_EP, dtype=jnp.int32) < cnt
    sp = jnp.where(valid, jnp.right_shift(comb, 16), fill_src)
    dp = jnp.where(valid, comb & 0xFFFF, fill_dst)
    ncha = (cnt + 127) // 128
    return (_interleave(sp), _interleave(dp),
            jnp.full((128,), ncha, jnp.int32))


def kernel(x, edge_index, mask, enc_mask_token, W_enc0, W_enc1, W_e2d, W_dec):
    src = edge_index[0]
    dst = edge_index[1]
    mf = mask.astype(jnp.float32)[:, None]

    ar = jnp.arange(_EP, dtype=jnp.int32)
    fill_src = (_N - 128) + (ar % 128)
    fill_dst = _N + (ar % 16)
    npad = _EP - _E
    sp_full = _interleave(jnp.concatenate([src, fill_src[:npad]]))
    dp_full = _interleave(jnp.concatenate([dst, fill_dst[:npad]]))
    cnt_full = jnp.full((128,), _NCHUNK, jnp.int32)
    sd2d = jnp.concatenate([src, fill_dst[:npad]]).reshape(_NCHUNK, 128)
    dd2d = jnp.concatenate([dst, fill_dst[:npad]]).reshape(_NCHUNK, 128)

    mi = mask.astype(jnp.int32)
    m_src = jnp.take(mi, src)
    m_dst = jnp.take(mi, dst)
    keep1 = 1 - m_src
    keep3 = keep1 * m_dst
    sp_1, dp_1, cnt_1 = _compact(keep1, src, dst, fill_src, fill_dst)
    sp_3, dp_3, cnt_3 = _compact(keep3, src, dst, fill_src, fill_dst)

    dsrc, ddst = _sc_degrees(sd2d, dd2d)
    t0a, t0b, t0c, t0d, ns, nd, s2 = _prep(x, mf, enc_mask_token, dsrc, ddst)
    a1 = _sc_propagate(t0a, t0b, t0c, t0d, sp_1, dp_1, cnt_1)
    t1 = _mm1(a1, nd, ns, W_enc0)
    a2 = _sc_propagate(*t1, sp_full, dp_full, cnt_full)
    t2 = _mm2(a2, nd, s2, W_enc1, W_e2d)
    a3 = _sc_propagate(*t2, sp_3, dp_3, cnt_3)
    loss = _final(a3, nd, W_dec, x, mf)
    return loss.reshape(())


# speedup vs baseline: 3.8358x; 3.8358x over previous
import functools

import jax
import jax.numpy as jnp
import numpy as np
from jax import lax
from jax.experimental import pallas as pl
from jax.experimental.pallas import tpu as pltpu
from jax.experimental.pallas import tpu_sc as plsc

_N = 10000
_NT = 10240
_E = 160000
_EP = 163840
_NCHUNK = 1280
_CPT = 80
_D = 512
_F = 128
_BR = 1000
_EPS = 1e-8

_SC_MESH = plsc.VectorSubcoreMesh(
    core_axis_name="c", subcore_axis_name="s", num_cores=2, num_subcores=16)


@functools.partial(
    pl.kernel,
    out_type=[jax.ShapeDtypeStruct((_NT, 16), jnp.float32),
              jax.ShapeDtypeStruct((_NT, 16), jnp.float32)],
    mesh=_SC_MESH,
    scratch_types=[
        pltpu.VMEM_SHARED((_NT, 16), jnp.float32),
        pltpu.VMEM((640, 16), jnp.float32),
        pltpu.VMEM((128, 16), jnp.float32),
        pltpu.VMEM((_CPT, 128), jnp.int32),
    ],
)
def _sc_degrees(sd_ref, dd_ref, dsrc_ref, ddst_ref, acc, zb, ones, idxv):
    c = lax.axis_index("c")
    s = lax.axis_index("s")

    def _zrow(i, _):
        zb[i, :] = jnp.zeros((16,), jnp.float32)
        return 0
    lax.fori_loop(0, 640, _zrow, 0)

    def _orow(i, _):
        ones[i, :] = jnp.full((16,), 1.0, jnp.float32)
        return 0
    lax.fori_loop(0, 128, _orow, 0)

    for half in range(2):
        idx_src = sd_ref if half == 0 else dd_ref
        out = dsrc_ref if half == 0 else ddst_ref

        @pl.when(c == half)
        def _():
            pltpu.sync_copy(zb, acc.at[pl.ds(s * 640, 640)])
            pltpu.sync_copy(idx_src.at[pl.ds(s * _CPT, _CPT)], idxv)

        plsc.subcore_barrier()

        @pl.when(c == half)
        def _():
            def _scat(j, _):
                pltpu.sync_copy(ones, acc.at[idxv.at[j]], add=True)
                return 0
            lax.fori_loop(0, _CPT, _scat, 0)

        plsc.subcore_barrier()

        @pl.when(c == half)
        def _():
            pltpu.sync_copy(acc.at[pl.ds(s * 640, 640)],
                            out.at[pl.ds(s * 640, 640)])


@functools.partial(
    pl.kernel,
    out_type=[jax.ShapeDtypeStruct((_NT, _F), jnp.float32)] * 4,
    mesh=_SC_MESH,
    scratch_types=[
        pltpu.VMEM_SHARED((_NT, _F), jnp.float32),
        pltpu.VMEM((_CPT * 128,), jnp.int32),
        pltpu.VMEM((4, 128), jnp.int32),
        pltpu.VMEM((2, 128, _F), jnp.float32),
        pltpu.VMEM((128,), jnp.int32),
        pltpu.SemaphoreType.DMA,
        pltpu.SemaphoreType.DMA,
        pltpu.SemaphoreType.DMA,
    ],
)
def _sc_propagate(t0, t1, t2, t3, sp_ref, dp_ref, cnt_ref,
                  o0, o1, o2, o3, acc, sidx, dring, rows, cntv,
                  gsem, dsem, ssem):
    c = lax.axis_index("c")
    s = lax.axis_index("s")

    pltpu.sync_copy(sp_ref.at[pl.ds(s * (_CPT * 128), _CPT * 128)], sidx)
    pltpu.sync_copy(cnt_ref, cntv)
    ncha = cntv[pl.ds(0, 16)][0]
    nloc = jnp.minimum(_CPT, jnp.maximum(0, (ncha - s + 15) // 16))

    tables = (t0, t1, t2, t3)
    outs = (o0, o1, o2, o3)
    for half in range(2):
        for b in range(2):
            tbl = tables[half * 2 + b]
            outb = outs[half * 2 + b]

            @pl.when(c == half)
            def _():
                def _zrow(i, _):
                    for k in range(8):
                        rows[0, i, pl.ds(k * 16, 16)] = (
                            jnp.zeros((16,), jnp.float32))
                    return 0
                lax.fori_loop(0, 128, _zrow, 0)
                for t in range(5):
                    pltpu.sync_copy(
                        rows.at[0], acc.at[pl.ds(s * 640 + t * 128, 128)])

            plsc.subcore_barrier()

            @pl.when(jnp.logical_and(c == half, nloc > 0))
            def _():
                @pl.when(nloc > 1)
                def _():
                    pltpu.async_copy(
                        dp_ref.at[pl.ds((s * _CPT + 1) * 128, 128)],
                        dring.at[1], dsem)

                pltpu.async_copy(
                    dp_ref.at[pl.ds(s * _CPT * 128, 128)], dring.at[0], dsem)
                pltpu.async_copy(
                    tbl.at[sidx.at[pl.ds(0, 128)]], rows.at[0], gsem)

                def _step(jj, _):
                    for bb in range(2):
                        cc = 2 * jj + bb

                        @pl.when(cc < nloc)
                        def _():
                            slot = lax.rem(cc, 4)

                            @pl.when(cc >= 1)
                            def _():
                                pltpu.make_async_copy(
                                    rows.at[1 - bb], acc.at[dring.at[slot]],
                                    ssem).wait()

                            @pl.when(cc + 1 < nloc)
                            def _():
                                pltpu.async_copy(
                                    tbl.at[sidx.at[pl.ds((cc + 1) * 128,
                                                         128)]],
                                    rows.at[1 - bb], gsem)

                            pltpu.make_async_copy(
                                dp_ref.at[pl.ds((s * _CPT + cc) * 128, 128)],
                                dring.at[slot], dsem).wait()
                            pltpu.make_async_copy(
                                tbl.at[sidx.at[pl.ds(cc * 128, 128)]],
                                rows.at[bb], gsem).wait()
                            pltpu.async_copy(
                                rows.at[bb], acc.at[dring.at[slot]],
                                ssem, add=True)

                            @pl.when(cc + 2 < nloc)
                            def _():
                                slot2 = lax.rem(cc + 2, 4)
                                pltpu.async_copy(
                                    dp_ref.at[pl.ds((s * _CPT + cc + 2) * 128,
                                                    128)],
                                    dring.at[slot2], dsem)
                    return 0
                lax.fori_loop(0, (nloc + 1) // 2, _step, 0)
                pltpu.make_async_copy(
                    rows.at[lax.rem(nloc - 1, 2)],
                    acc.at[dring.at[lax.rem(nloc - 1, 4)]], ssem).wait()

            plsc.subcore_barrier()

            @pl.when(c == half)
            def _():
                pltpu.sync_copy(acc.at[pl.ds(s * 640, 640)],
                                outb.at[pl.ds(s * 640, 640)])

            plsc.subcore_barrier()


def _row(i):
    return (i, 0)


def _const(i):
    return (0, 0)


def _prep_body(x_ref, m_ref, tok_ref, ds_ref, dd_ref,
               t_refs0, t_refs1, t_refs2, t_refs3, ns_ref, nd_ref, s2_ref):
    ns = lax.rsqrt(jnp.maximum(ds_ref[:, 0:1], 1.0))
    nd = lax.rsqrt(jnp.maximum(dd_ref[:, 0:1], 1.0))
    m = m_ref[...]
    xm = jnp.where(m > 0.5, tok_ref[...], x_ref[...])
    t0 = xm * ns
    for k, ref in enumerate((t_refs0, t_refs1, t_refs2, t_refs3)):
        ref[...] = t0[:, k * _F:(k + 1) * _F]
    ns_ref[...] = ns
    nd_ref[...] = nd
    s2_ref[...] = jnp.where(m > 0.5, 0.0, ns)


def _prep(x, mf, tok, dsrc, ddst):
    return pl.pallas_call(
        _prep_body,
        grid=(_N // _BR,),
        in_specs=[
            pl.BlockSpec((_BR, _D), _row),
            pl.BlockSpec((_BR, 1), _row),
            pl.BlockSpec((1, _D), _const),
            pl.BlockSpec((_BR, 16), _row),
            pl.BlockSpec((_BR, 16), _row),
        ],
        out_specs=[pl.BlockSpec((_BR, _F), _row)] * 4 + [
            pl.BlockSpec((_BR, 1), _row)] * 3,
        out_shape=[jax.ShapeDtypeStruct((_N, _F), jnp.float32)] * 4 + [
            jax.ShapeDtypeStruct((_N, 1), jnp.float32)] * 3,
    )(x, mf, tok, dsrc, ddst)


def _mm1_body(a0, a1, a2, a3, nd_ref, ns_ref, w_ref,
              o0, o1, o2, o3):
    a = jnp.concatenate([a0[...], a1[...], a2[...], a3[...]], axis=1)
    h = jnp.dot(a * nd_ref[...], w_ref[...],
                preferred_element_type=jnp.float32)
    t = jax.nn.relu(h) * ns_ref[...]
    for k, ref in enumerate((o0, o1, o2, o3)):
        ref[...] = t[:, k * _F:(k + 1) * _F]


def _mm1(aggs, nd, ns, w):
    return pl.pallas_call(
        _mm1_body,
        grid=(_N // _BR,),
        in_specs=[pl.BlockSpec((_BR, _F), _row)] * 4 + [
            pl.BlockSpec((_BR, 1), _row),
            pl.BlockSpec((_BR, 1), _row),
            pl.BlockSpec((_D, _D), _const),
        ],
        out_specs=[pl.BlockSpec((_BR, _F), _row)] * 4,
        out_shape=[jax.ShapeDtypeStruct((_N, _F), jnp.float32)] * 4,
    )(*aggs, nd, ns, w)


def _mm2_body(a0, a1, a2, a3, nd_ref, s2_ref, w1_ref, w2_ref,
              o0, o1, o2, o3):
    a = jnp.concatenate([a0[...], a1[...], a2[...], a3[...]], axis=1)
    h = jnp.dot(a * nd_ref[...], w1_ref[...],
                preferred_element_type=jnp.float32)
    h = jax.nn.relu(h)
    rep = jnp.dot(h, w2_ref[...], preferred_element_type=jnp.float32)
    t = rep * s2_ref[...]
    for k, ref in enumerate((o0, o1, o2, o3)):
        ref[...] = t[:, k * _F:(k + 1) * _F]


def _mm2(aggs, nd, s2, w1, w2):
    return pl.pallas_call(
        _mm2_body,
        grid=(_N // _BR,),
        in_specs=[pl.BlockSpec((_BR, _F), _row)] * 4 + [
            pl.BlockSpec((_BR, 1), _row),
            pl.BlockSpec((_BR, 1), _row),
            pl.BlockSpec((_D, _D), _const),
            pl.BlockSpec((_D, _D), _const),
        ],
        out_specs=[pl.BlockSpec((_BR, _F), _row)] * 4,
        out_shape=[jax.ShapeDtypeStruct((_N, _F), jnp.float32)] * 4,
    )(*aggs, nd, s2, w1, w2)


def _final_body(a0, a1, a2, a3, nd_ref, w_ref, x_ref, m_ref, o_ref, acc_ref):
    i = pl.program_id(0)

    @pl.when(i == 0)
    def _():
        acc_ref[0] = 0.0
        acc_ref[1] = 0.0

    a = jnp.concatenate([a0[...], a1[...], a2[...], a3[...]], axis=1)
    r = jnp.dot(a * nd_ref[...], w_ref[...],
                preferred_element_type=jnp.float32)
    x = x_ref[...]
    xr = jnp.sum(x * r, axis=1, keepdims=True)
    xx = jnp.sum(x * x, axis=1, keepdims=True)
    rr = jnp.sum(r * r, axis=1, keepdims=True)
    cos = xr / ((jnp.sqrt(xx) + _EPS) * (jnp.sqrt(rr) + _EPS))
    per = (1.0 - cos) ** 2
    m = m_ref[...]
    acc_ref[0] += jnp.sum(per * m)
    acc_ref[1] += jnp.sum(m)

    @pl.when(i == pl.num_programs(0) - 1)
    def _():
        o_ref[...] = jnp.reshape(acc_ref[0] / jnp.maximum(acc_ref[1], 1.0),
                                 (1, 1))


def _final(aggs, nd, w, x, mf):
    return pl.pallas_call(
        _final_body,
        grid=(_N // _BR,),
        in_specs=[pl.BlockSpec((_BR, _F), _row)] * 4 + [
            pl.BlockSpec((_BR, 1), _row),
            pl.BlockSpec((_D, _D), _const),
            pl.BlockSpec((_BR, _D), _row),
            pl.BlockSpec((_BR, 1), _row),
        ],
        out_specs=pl.BlockSpec((1, 1), _const),
        out_shape=jax.ShapeDtypeStruct((1, 1), jnp.float32),
        scratch_shapes=[pltpu.SMEM((2,), jnp.float32)],
    )(*aggs, nd, w, x, mf)


_KMAP = np.array([(d % _CPT) * 16 + d // _CPT for d in range(_NCHUNK)],
                 dtype=np.int32)


def _interleave(a):
    return a.reshape(_NCHUNK, 128)[_KMAP].reshape(-1)


_ER = _E // 128
_TRIU128 = np.triu(np.ones((128, 128), np.float32))
_TRIL_R = np.tril(np.ones((_ER, _ER), np.float32))


def _compact(keep, src, dst, fill_src, fill_dst):
    kb = keep > 0.5
    k2 = kb.astype(jnp.float32).reshape(_ER, 128)
    within = jnp.dot(k2, _TRIU128)
    rowsums = within[:, 127]
    rowincl = jnp.dot(_TRIL_R, rowsums)
    pos_incl = ((rowincl - rowsums)[:, None] + within).astype(
        jnp.int32).reshape(-1)
    cnt = rowincl[_ER - 1].astype(jnp.int32)
    idx = jnp.where(kb, pos_incl - 1, _EP)
    comb = jnp.zeros((_EP,), jnp.int32).at[idx].add(
        jnp.left_shift(src, 16) | dst, mode='drop')
    valid = jnp.arange(_EP, dtype=jnp.int32) < cnt
    sp = jnp.where(valid, jnp.right_shift(comb, 16), fill_src)
    dp = jnp.where(valid, comb & 0xFFFF, fill_dst)
    ncha = (cnt + 127) // 128
    return (_interleave(sp), _interleave(dp),
            jnp.full((128,), ncha, jnp.int32))


def kernel(x, edge_index, mask, enc_mask_token, W_enc0, W_enc1, W_e2d, W_dec):
    src = edge_index[0]
    dst = edge_index[1]
    mf = mask.astype(jnp.float32)[:, None]

    ar = jnp.arange(_EP, dtype=jnp.int32)
    fill_src = (_N - 128) + (ar % 128)
    fill_dst = _N + (ar % 16)
    npad = _EP - _E
    sp_full = _interleave(jnp.concatenate([src, fill_src[:npad]]))
    dp_full = _interleave(jnp.concatenate([dst, fill_dst[:npad]]))
    cnt_full = jnp.full((128,), _NCHUNK, jnp.int32)
    sd2d = jnp.concatenate([src, fill_dst[:npad]]).reshape(_NCHUNK, 128)
    dd2d = jnp.concatenate([dst, fill_dst[:npad]]).reshape(_NCHUNK, 128)

    mpad = jnp.zeros((1 << 24,), jnp.float32).at[: _N].set(mf[:, 0])
    m_src = jnp.take(mpad, src)
    m_dst = jnp.take(mpad, dst)
    keep1 = 1.0 - m_src
    keep3 = keep1 * m_dst
    dsrc, ddst = _sc_degrees(sd2d, dd2d)
    sp_1, dp_1, cnt_1 = _compact(keep1, src, dst, fill_src, fill_dst)
    sp_3, dp_3, cnt_3 = _compact(keep3, src, dst, fill_src, fill_dst)
    t0a, t0b, t0c, t0d, ns, nd, s2 = _prep(x, mf, enc_mask_token, dsrc, ddst)
    a1 = _sc_propagate(t0a, t0b, t0c, t0d, sp_1, dp_1, cnt_1)
    t1 = _mm1(a1, nd, ns, W_enc0)
    a2 = _sc_propagate(*t1, sp_full, dp_full, cnt_full)
    t2 = _mm2(a2, nd, s2, W_enc1, W_e2d)
    a3 = _sc_propagate(*t2, sp_3, dp_3, cnt_3)
    loss = _final(a3, nd, W_dec, x, mf)
    return loss.reshape(())
